# baseline (speedup 1.0000x reference)
import functools

import jax
import jax.numpy as jnp
from jax import lax
from jax.experimental import pallas as pl
from jax.experimental.pallas import tpu as pltpu
from jax.experimental.pallas import tpu_sc as plsc

BATCH = 16384
NTILES = 16
PER_TILE = BATCH // NTILES
CHUNK = 128
NCHUNK = PER_TILE // CHUNK
L = 16
NVEC = PER_TILE // L
EPS = 1e-5
EMB_ROWS = 1000000
TABLE_PAD = 1000448


def _rsqrt16(x):
    i = lax.bitcast_convert_type(x, jnp.int32)
    i = jnp.int32(0x5F3759DF) - lax.shift_right_arithmetic(i, 1)
    y = lax.bitcast_convert_type(i, jnp.float32)
    half = x * 0.5
    for _ in range(4):
        y = y * (1.5 - half * y * y)
    return y


def _lane_sum_bcast(v):
    idx0 = lax.iota(jnp.int32, 16)
    dnums = lax.GatherDimensionNumbers(
        offset_dims=(), collapsed_slice_dims=(0,), start_index_map=(0,)
    )
    for s in (1, 2, 4, 8):
        perm = lax.gather(
            v, (idx0 ^ s)[:, None], dnums, slice_sizes=(1,),
            mode=lax.GatherScatterMode.PROMISE_IN_BOUNDS,
        )
        v = v + perm
    return v


def _embedder_body(x_hbm, table_hbm, params_hbm, out_hbm,
                   idx_v, rows_v, params_v, stage_v, all_v, out_v,
                   shared_sums, sem):
    tid = lax.axis_index("s")

    pltpu.sync_copy(x_hbm.at[pl.ds(tid * PER_TILE, PER_TILE)], idx_v)

    def _fire(j, c):
        pltpu.async_copy(
            table_hbm.at[idx_v.at[pl.ds(j * CHUNK, CHUNK)]],
            rows_v.at[pl.ds(j * CHUNK, CHUNK)],
            sem,
        )
        return c

    lax.fori_loop(0, NCHUNK, _fire, 0)
    pltpu.sync_copy(params_hbm, params_v)

    def _drain(j, c):
        pltpu.make_async_copy(
            table_hbm.at[idx_v.at[pl.ds(j * CHUNK, CHUNK)]],
            rows_v.at[pl.ds(j * CHUNK, CHUNK)],
            sem,
        ).wait()
        return c

    lax.fori_loop(0, NCHUNK, _drain, 0)

    def _stats_step(i, carry):
        s, q = carry
        v = rows_v[pl.ds(i * L, L)]
        return s + v, q + v * v

    zero = jnp.zeros((L,), jnp.float32)
    acc_s, acc_q = lax.fori_loop(0, NVEC, _stats_step, (zero, zero))
    stage_v[0, :] = acc_s
    stage_v[1, :] = acc_q

    pltpu.sync_copy(stage_v, shared_sums.at[tid])
    plsc.subcore_barrier()
    pltpu.sync_copy(shared_sums, all_v)

    def _reduce_step(t, carry):
        s, q = carry
        return s + all_v[t, 0, :], q + all_v[t, 1, :]

    tot_s, tot_q = lax.fori_loop(0, NTILES, _reduce_step, (zero, zero))
    sum_e = _lane_sum_bcast(tot_s)
    sum_q = _lane_sum_bcast(tot_q)

    w = params_v[0, :]
    bias = params_v[1, :]
    bn_g = params_v[2, :]
    bn_b = params_v[3, :]
    ln_g = params_v[4, :]
    ln_b = params_v[5, :]

    inv_b = 1.0 / BATCH
    mean_e = sum_e * inv_b
    var_e = sum_q * inv_b - mean_e * mean_e
    mu = w * mean_e + bias
    var = w * w * var_e
    inv_sigma = _rsqrt16(var + EPS)
    scale = inv_sigma * bn_g
    shift = bn_b - mu * scale

    def _norm_step(i, carry):
        v = rows_v[pl.ds(i * L, L)]
        lin = v * w + bias
        bn = lin * scale + shift
        m = bn
        d = bn - m
        v_ln = d * d
        out_v[pl.ds(i * L, L)] = d * _rsqrt16(v_ln + EPS) * ln_g + ln_b
        return carry

    lax.fori_loop(0, NVEC, _norm_step, 0)

    pltpu.sync_copy(out_v, out_hbm.at[pl.ds(tid * PER_TILE, PER_TILE)])


@jax.jit
def _embedder_sc(x1d, table1d, params):
    mesh = plsc.VectorSubcoreMesh(
        core_axis_name="c", subcore_axis_name="s", num_cores=1
    )
    return pl.kernel(
        _embedder_body,
        out_type=jax.ShapeDtypeStruct((BATCH,), jnp.float32),
        mesh=mesh,
        scratch_types=[
            pltpu.VMEM((PER_TILE,), jnp.int32),
            pltpu.VMEM((PER_TILE,), jnp.float32),
            pltpu.VMEM((6, L), jnp.float32),
            pltpu.VMEM((2, L), jnp.float32),
            pltpu.VMEM((NTILES, 2, L), jnp.float32),
            pltpu.VMEM((PER_TILE,), jnp.float32),
            pltpu.VMEM_SHARED((NTILES, 2, L), jnp.float32),
            pltpu.SemaphoreType.DMA,
        ],
    )(x1d, table1d, params)


def kernel(x, table, W, b, bn_gamma, bn_beta, ln_gamma, ln_beta):
    tpad = jnp.pad(table, ((0, TABLE_PAD - EMB_ROWS), (0, 0)))
    table1d = tpad.reshape(-1)
    scal = jnp.stack([
        W.reshape(()), b.reshape(()),
        bn_gamma.reshape(()), bn_beta.reshape(()),
        ln_gamma.reshape(()), ln_beta.reshape(()),
    ]).astype(jnp.float32)
    params = jnp.broadcast_to(scal[:, None], (6, L))
    out = _embedder_sc(x, table1d, params)
    return out.reshape(BATCH, 1)


# speedup vs baseline: 1.0092x; 1.0092x over previous
import functools

import jax
import jax.numpy as jnp
from jax import lax
from jax.experimental import pallas as pl
from jax.experimental.pallas import tpu as pltpu
from jax.experimental.pallas import tpu_sc as plsc

BATCH = 16384
NTILES = 16
PER_TILE = BATCH // NTILES
CHUNK = 128
NCHUNK = PER_TILE // CHUNK
L = 16
NVEC = PER_TILE // L
EPS = 1e-5
EMB_ROWS = 1000000
TABLE_PAD = 1000448


def _rsqrt16(x, newton_steps=4):
    i = lax.bitcast_convert_type(x, jnp.int32)
    i = jnp.int32(0x5F3759DF) - lax.shift_right_arithmetic(i, 1)
    y = lax.bitcast_convert_type(i, jnp.float32)
    half = x * 0.5
    for _ in range(newton_steps):
        y = y * (1.5 - half * y * y)
    return y


def _lane_sum_bcast(v):
    idx0 = lax.iota(jnp.int32, 16)
    dnums = lax.GatherDimensionNumbers(
        offset_dims=(), collapsed_slice_dims=(0,), start_index_map=(0,)
    )
    for s in (1, 2, 4, 8):
        perm = lax.gather(
            v, (idx0 ^ s)[:, None], dnums, slice_sizes=(1,),
            mode=lax.GatherScatterMode.PROMISE_IN_BOUNDS,
        )
        v = v + perm
    return v


def _embedder_body(x_hbm, table_hbm, params_hbm, out_hbm,
                   idx_v, rows_v, params_v, stage_v, all_v, out_v,
                   shared_sums, sem):
    tid = lax.axis_index("s")

    pltpu.sync_copy(x_hbm.at[pl.ds(tid * PER_TILE, PER_TILE)], idx_v)
    copies = []
    for j in range(NCHUNK):
        copies.append(
            pltpu.async_copy(
                table_hbm.at[idx_v.at[pl.ds(j * CHUNK, CHUNK)]],
                rows_v.at[pl.ds(j * CHUNK, CHUNK)],
                sem,
            )
        )
    pltpu.sync_copy(params_hbm, params_v)

    def _stats_step(i, carry):
        s, q = carry
        v = rows_v[pl.ds(i * L, L)]
        return s + v, q + v * v

    zero = jnp.zeros((L,), jnp.float32)
    acc_s, acc_q = zero, zero
    vec_per_chunk = CHUNK // L
    for j in range(NCHUNK):
        copies[j].wait()
        acc_s, acc_q = lax.fori_loop(
            j * vec_per_chunk, (j + 1) * vec_per_chunk,
            _stats_step, (acc_s, acc_q))
    stage_v[0, :] = acc_s
    stage_v[1, :] = acc_q

    pltpu.sync_copy(stage_v, shared_sums.at[tid])
    plsc.subcore_barrier()
    pltpu.sync_copy(shared_sums, all_v)

    def _reduce_step(t, carry):
        s, q = carry
        return s + all_v[t, 0, :], q + all_v[t, 1, :]

    tot_s, tot_q = lax.fori_loop(0, NTILES, _reduce_step, (zero, zero))
    sum_e = _lane_sum_bcast(tot_s)
    sum_q = _lane_sum_bcast(tot_q)

    w = params_v[0, :]
    bias = params_v[1, :]
    bn_g = params_v[2, :]
    bn_b = params_v[3, :]
    ln_g = params_v[4, :]
    ln_b = params_v[5, :]

    inv_b = 1.0 / BATCH
    mean_e = sum_e * inv_b
    var_e = sum_q * inv_b - mean_e * mean_e
    mu = w * mean_e + bias
    var = w * w * var_e
    inv_sigma = _rsqrt16(var + EPS)
    scale = inv_sigma * bn_g
    shift = bn_b - mu * scale

    def _norm_step(i, carry):
        v = rows_v[pl.ds(i * L, L)]
        lin = v * w + bias
        bn = lin * scale + shift
        m = bn
        d = bn - m
        v_ln = d * d
        out_v[pl.ds(i * L, L)] = d * _rsqrt16(v_ln + EPS, 2) * ln_g + ln_b
        return carry

    lax.fori_loop(0, NVEC, _norm_step, 0)

    pltpu.sync_copy(out_v, out_hbm.at[pl.ds(tid * PER_TILE, PER_TILE)])


@jax.jit
def _embedder_sc(x1d, table1d, params):
    mesh = plsc.VectorSubcoreMesh(
        core_axis_name="c", subcore_axis_name="s", num_cores=1
    )
    return pl.kernel(
        _embedder_body,
        out_type=jax.ShapeDtypeStruct((BATCH,), jnp.float32),
        mesh=mesh,
        scratch_types=[
            pltpu.VMEM((PER_TILE,), jnp.int32),
            pltpu.VMEM((PER_TILE,), jnp.float32),
            pltpu.VMEM((6, L), jnp.float32),
            pltpu.VMEM((2, L), jnp.float32),
            pltpu.VMEM((NTILES, 2, L), jnp.float32),
            pltpu.VMEM((PER_TILE,), jnp.float32),
            pltpu.VMEM_SHARED((NTILES, 2, L), jnp.float32),
            pltpu.SemaphoreType.DMA,
        ],
    )(x1d, table1d, params)


def kernel(x, table, W, b, bn_gamma, bn_beta, ln_gamma, ln_beta):
    tpad = jnp.pad(table, ((0, TABLE_PAD - EMB_ROWS), (0, 0)))
    table1d = tpad.reshape(-1)
    scal = jnp.stack([
        W.reshape(()), b.reshape(()),
        bn_gamma.reshape(()), bn_beta.reshape(()),
        ln_gamma.reshape(()), ln_beta.reshape(()),
    ]).astype(jnp.float32)
    params = jnp.broadcast_to(scal[:, None], (6, L))
    out = _embedder_sc(x, table1d, params)
    return out.reshape(BATCH, 1)
